# Initial kernel scaffold; baseline (speedup 1.0000x reference)
import jax
import jax.numpy as jnp
from jax.experimental import pallas as pl


def kernel(input, index, value):
    raise NotImplementedError("write your pallas kernel here")


# speedup vs baseline: 4.7998x; 4.7998x over previous
import functools

import jax
import jax.numpy as jnp
from jax import lax
from jax.experimental import pallas as pl
from jax.experimental.pallas import tpu as pltpu
from jax.experimental.pallas import tpu_sc as plsc

NC = 2
NS = 16
NW = NC * NS
L = 16

BCH = 10000
UNROLL = 5


def _make_kernel(M, B, dtype):
    base_sz = (M // NW) // 8 * 8
    last_sz = M - (NW - 1) * base_sz
    assert last_sz % 8 == 0 and last_sz >= base_sz
    n_chunks = B // BCH
    assert B % BCH == 0 and BCH % (L * UNROLL) == 0

    mesh = plsc.VectorSubcoreMesh(
        core_axis_name="c", subcore_axis_name="s", num_cores=NC, num_subcores=NS
    )

    @functools.partial(
        pl.kernel,
        out_type=jax.ShapeDtypeStruct((M,), dtype),
        mesh=mesh,
        scratch_types=[
            pltpu.VMEM((last_sz,), jnp.int32),
            pltpu.VMEM((BCH,), jnp.int32),
            pltpu.VMEM((BCH,), jnp.int32),
        ],
        compiler_params=pltpu.CompilerParams(needs_layout_passes=False),
    )
    def scatter_kernel(in_hbm, idx_hbm, val_hbm, out_hbm, local, idxb, valb):
        wid = lax.axis_index("s") * NC + lax.axis_index("c")
        base = wid * base_sz
        is_last = wid == NW - 1
        n_local = jnp.where(is_last, last_sz, base_sz)
        vbase = jnp.full((L,), base, jnp.int32)
        vend = jnp.full((L,), base + n_local, jnp.int32)

        @pl.when(jnp.logical_not(is_last))
        def _():
            pltpu.sync_copy(in_hbm.at[pl.ds(base, base_sz)],
                            local.at[pl.ds(0, base_sz)])

        @pl.when(is_last)
        def _():
            pltpu.sync_copy(in_hbm.at[pl.ds(base, last_sz)], local)

        for c in range(n_chunks):
            pltpu.sync_copy(idx_hbm.at[pl.ds(c * BCH, BCH)], idxb)
            pltpu.sync_copy(val_hbm.at[pl.ds(c * BCH, BCH)], valb)

            def body(j, carry):
                for u in range(UNROLL):
                    off = pl.multiple_of(j * (L * UNROLL) + u * L, L)
                    idx = idxb[pl.ds(off, L)]
                    val = valb[pl.ds(off, L)]
                    mask = jnp.logical_and(idx >= vbase, idx < vend)
                    loc = jnp.where(mask, idx - vbase, 0)
                    plsc.store_scatter(local, [loc], val, mask=mask)
                return carry

            lax.fori_loop(0, BCH // (L * UNROLL), body, 0)

        @pl.when(jnp.logical_not(is_last))
        def _():
            pltpu.sync_copy(local.at[pl.ds(0, base_sz)],
                            out_hbm.at[pl.ds(base, base_sz)])

        @pl.when(is_last)
        def _():
            pltpu.sync_copy(local, out_hbm.at[pl.ds(base, last_sz)])

    return scatter_kernel


@jax.jit
def kernel(input, index, value):
    M = input.shape[0]
    B = index.shape[0]
    out = _make_kernel(M, B, input.dtype)(
        input.astype(jnp.int32), index.astype(jnp.int32), value.astype(jnp.int32)
    )
    return out


# speedup vs baseline: 6.1483x; 1.2810x over previous
import functools

import jax
import jax.numpy as jnp
from jax import lax
from jax.experimental import pallas as pl
from jax.experimental.pallas import tpu as pltpu
from jax.experimental.pallas import tpu_sc as plsc

NC = 2
NS = 16
NW = NC * NS
L = 16

BCH = 20000
UNROLL = 10


def _make_kernel(M, B, dtype):
    base_sz = (M // NW) // 8 * 8
    last_sz = M - (NW - 1) * base_sz
    assert last_sz % 8 == 0 and last_sz >= base_sz
    n_chunks = B // BCH
    assert B % BCH == 0 and BCH % (L * UNROLL) == 0

    mesh = plsc.VectorSubcoreMesh(
        core_axis_name="c", subcore_axis_name="s", num_cores=NC, num_subcores=NS
    )

    @functools.partial(
        pl.kernel,
        out_type=jax.ShapeDtypeStruct((M,), dtype),
        mesh=mesh,
        scratch_types=[
            pltpu.VMEM((last_sz,), jnp.int32),
            pltpu.VMEM((BCH,), jnp.int32),
            pltpu.VMEM((BCH,), jnp.int32),
            pltpu.VMEM((BCH,), jnp.int32),
            pltpu.VMEM((BCH,), jnp.int32),
            pltpu.SemaphoreType.DMA,
            pltpu.SemaphoreType.DMA,
        ],
        compiler_params=pltpu.CompilerParams(needs_layout_passes=False),
    )
    def scatter_kernel(in_hbm, idx_hbm, val_hbm, out_hbm,
                       local, idxb0, valb0, idxb1, valb1, sem0, sem1):
        idxbufs = [idxb0, idxb1]
        valbufs = [valb0, valb1]
        sems = [sem0, sem1]
        wid = lax.axis_index("s") * NC + lax.axis_index("c")
        base = wid * base_sz
        is_last = wid == NW - 1
        n_local = jnp.where(is_last, last_sz, base_sz)
        vbase = jnp.full((L,), base, jnp.int32)
        vn = jnp.full((L,), n_local, jnp.uint32)

        def start_fetch(c):
            slot = c % 2
            pltpu.async_copy(idx_hbm.at[pl.ds(c * BCH, BCH)], idxbufs[slot],
                             sems[slot])
            pltpu.async_copy(val_hbm.at[pl.ds(c * BCH, BCH)], valbufs[slot],
                             sems[slot])

        def wait_fetch(c):
            slot = c % 2
            pltpu.make_async_copy(idx_hbm.at[pl.ds(c * BCH, BCH)],
                                  idxbufs[slot], sems[slot]).wait()
            pltpu.make_async_copy(val_hbm.at[pl.ds(c * BCH, BCH)],
                                  valbufs[slot], sems[slot]).wait()

        start_fetch(0)

        @pl.when(jnp.logical_not(is_last))
        def _():
            pltpu.sync_copy(in_hbm.at[pl.ds(base, base_sz)],
                            local.at[pl.ds(0, base_sz)])

        @pl.when(is_last)
        def _():
            pltpu.sync_copy(in_hbm.at[pl.ds(base, last_sz)], local)

        for c in range(n_chunks):
            wait_fetch(c)
            if c + 1 < n_chunks:
                start_fetch(c + 1)
            idxb = idxbufs[c % 2]
            valb = valbufs[c % 2]

            def body(j, carry):
                for u in range(UNROLL):
                    off = pl.multiple_of(j * (L * UNROLL) + u * L, L)
                    idx = idxb[pl.ds(off, L)]
                    val = valb[pl.ds(off, L)]
                    loc = idx - vbase
                    mask = plsc.bitcast(loc, jnp.uint32) < vn
                    plsc.store_scatter(local, [loc], val, mask=mask)
                return carry

            lax.fori_loop(0, BCH // (L * UNROLL), body, 0)

        @pl.when(jnp.logical_not(is_last))
        def _():
            pltpu.sync_copy(local.at[pl.ds(0, base_sz)],
                            out_hbm.at[pl.ds(base, base_sz)])

        @pl.when(is_last)
        def _():
            pltpu.sync_copy(local, out_hbm.at[pl.ds(base, last_sz)])

    return scatter_kernel


@jax.jit
def kernel(input, index, value):
    M = input.shape[0]
    B = index.shape[0]
    out = _make_kernel(M, B, input.dtype)(
        input.astype(jnp.int32), index.astype(jnp.int32), value.astype(jnp.int32)
    )
    return out


# speedup vs baseline: 9.9500x; 1.6183x over previous
import functools

import jax
import jax.numpy as jnp
from jax import lax
from jax.experimental import pallas as pl
from jax.experimental.pallas import tpu as pltpu
from jax.experimental.pallas import tpu_sc as plsc

NC = 2
NS = 16
NW = NC * NS
L = 16

BCH = 20000
UNROLL = 10


def _make_kernel(M, B, dtype):
    base_sz = (M // NW) // 8 * 8
    last_sz = M - (NW - 1) * base_sz
    assert last_sz % 8 == 0 and last_sz >= base_sz
    n_chunks = B // BCH
    assert B % BCH == 0 and BCH % (L * UNROLL) == 0

    mesh = plsc.VectorSubcoreMesh(
        core_axis_name="c", subcore_axis_name="s", num_cores=NC, num_subcores=NS
    )

    @functools.partial(
        pl.kernel,
        out_type=jax.ShapeDtypeStruct((M,), dtype),
        mesh=mesh,
        scratch_types=[
            pltpu.VMEM((last_sz,), jnp.int32),
            pltpu.VMEM((BCH,), jnp.int32),
            pltpu.VMEM((BCH,), jnp.int32),
            pltpu.VMEM((BCH,), jnp.int32),
            pltpu.VMEM((BCH,), jnp.int32),
            pltpu.SemaphoreType.DMA,
            pltpu.SemaphoreType.DMA,
        ],
        compiler_params=pltpu.CompilerParams(needs_layout_passes=False),
    )
    def scatter_kernel(in_hbm, idx_hbm, val_hbm, out_hbm,
                       local, idxb0, valb0, idxb1, valb1, sem0, sem1):
        idxbufs = [idxb0, idxb1]
        valbufs = [valb0, valb1]
        sems = [sem0, sem1]
        wid = lax.axis_index("s") * NC + lax.axis_index("c")
        base = wid * base_sz
        is_last = wid == NW - 1
        n_local = jnp.where(is_last, last_sz, base_sz)
        vbase = jnp.full((L,), base, jnp.int32)
        vn = jnp.full((L,), n_local, jnp.uint32)

        def start_fetch(c):
            slot = c % 2
            pltpu.async_copy(idx_hbm.at[pl.ds(c * BCH, BCH)], idxbufs[slot],
                             sems[slot])
            pltpu.async_copy(val_hbm.at[pl.ds(c * BCH, BCH)], valbufs[slot],
                             sems[slot])

        def wait_fetch(c):
            slot = c % 2
            pltpu.make_async_copy(idx_hbm.at[pl.ds(c * BCH, BCH)],
                                  idxbufs[slot], sems[slot]).wait()
            pltpu.make_async_copy(val_hbm.at[pl.ds(c * BCH, BCH)],
                                  valbufs[slot], sems[slot]).wait()

        start_fetch(0)

        @pl.when(jnp.logical_not(is_last))
        def _():
            pltpu.sync_copy(in_hbm.at[pl.ds(base, base_sz)],
                            local.at[pl.ds(0, base_sz)])

        @pl.when(is_last)
        def _():
            pltpu.sync_copy(in_hbm.at[pl.ds(base, last_sz)], local)

        for c in range(n_chunks):
            wait_fetch(c)
            if c + 1 < n_chunks:
                start_fetch(c + 1)
            idxb = idxbufs[c % 2]
            valb = valbufs[c % 2]

            def body(j, carry):
                locs, vals, masks = [], [], []
                for u in range(UNROLL):
                    off = pl.multiple_of(j * (L * UNROLL) + u * L, L)
                    idx = idxb[pl.ds(off, L)]
                    vals.append(valb[pl.ds(off, L)])
                    loc = idx - vbase
                    locs.append(loc)
                    masks.append(plsc.bitcast(loc, jnp.uint32) < vn)
                for u in range(UNROLL):
                    plsc.store_scatter(local, [locs[u]], vals[u], mask=masks[u])
                return carry

            lax.fori_loop(0, BCH // (L * UNROLL), body, 0)

        @pl.when(jnp.logical_not(is_last))
        def _():
            pltpu.sync_copy(local.at[pl.ds(0, base_sz)],
                            out_hbm.at[pl.ds(base, base_sz)])

        @pl.when(is_last)
        def _():
            pltpu.sync_copy(local, out_hbm.at[pl.ds(base, last_sz)])

    return scatter_kernel


@jax.jit
def kernel(input, index, value):
    M = input.shape[0]
    B = index.shape[0]
    out = _make_kernel(M, B, input.dtype)(
        input.astype(jnp.int32), index.astype(jnp.int32), value.astype(jnp.int32)
    )
    return out
